# Initial kernel scaffold; baseline (speedup 1.0000x reference)
import jax
import jax.numpy as jnp
from jax.experimental import pallas as pl


def kernel(x, edge_index, edge_attr, u, batch, W1, att_src1, att_dst1, We1, att_edge1, b1, W2, att_src2, att_dst2, We2, att_edge2, b2, m1w, m1b, m2w, m2b, m3w, m3b):
    raise NotImplementedError("write your pallas kernel here")


# speedup vs baseline: 1.1242x; 1.1242x over previous
import functools

import jax
import jax.numpy as jnp
from jax.experimental import pallas as pl

N = 10000
E = 320000
DF = 128
DE = 16
DG = 16
G = 100
H = 8
C = 64


def _leaky(x):
    return jnp.maximum(x, 0.2 * x)


def _mlp_pool_kernel(h_ref, bsel_ref, usel_ref, w1_ref, b1_ref, w2_ref, b2_ref,
                     w3_ref, b3_ref, out_ref):
    ge = bsel_ref[...] @ h_ref[...]
    comb = jnp.concatenate([ge, usel_ref[...]], axis=1)
    z = jnp.maximum(comb @ w1_ref[...] + b1_ref[...], 0.0)
    z = jnp.maximum(z @ w2_ref[...] + b2_ref[...], 0.0)
    out_ref[...] = z @ w3_ref[...] + b3_ref[...]


def _gat_layer(x_in, src, dst, edge_attr, loop_attr, W, att_src, att_dst, We,
               att_edge, bias, concat):
    Vs = jnp.einsum("fhc,hc->fh", W.reshape(-1, H, C), att_src[0])
    Vd = jnp.einsum("fhc,hc->fh", W.reshape(-1, H, C), att_dst[0])
    Ve = jnp.einsum("dhc,hc->dh", We.reshape(DE, H, C), att_edge[0])

    h = x_in @ W
    al_s = x_in @ Vs
    al_d = x_in @ Vd
    al_e = edge_attr @ Ve
    al_e_loop = loop_attr @ Ve

    M = al_s.max(0) + jnp.maximum(al_e.max(0), al_e_loop.max(0))
    stab = _leaky(al_d + M[None, :])

    t = al_s[src] + al_d[dst] + al_e
    n = jnp.exp(_leaky(t) - stab[dst])
    t_self = al_s + al_d + al_e_loop
    n_self = jnp.exp(_leaky(t_self) - stab)

    denom = jax.ops.segment_sum(n, dst, num_segments=N) + n_self
    h3 = h.reshape(N, H, C)
    outs = jax.ops.segment_sum(h3[src] * n[:, :, None], dst, num_segments=N)
    outs = outs + h3 * n_self[:, :, None]
    out = outs / denom[:, :, None]
    if concat:
        return out.reshape(N, H * C) + bias
    return out.mean(axis=1) + bias


def kernel(x, edge_index, edge_attr, u, batch, W1, att_src1, att_dst1, We1,
           att_edge1, b1, W2, att_src2, att_dst2, We2, att_edge2, b2, m1w,
           m1b, m2w, m2b, m3w, m3b):
    src, dst = edge_index[0], edge_index[1]

    deg = jax.ops.segment_sum(jnp.ones((E,), jnp.float32), dst, num_segments=N)
    attr_sum = jax.ops.segment_sum(edge_attr, dst, num_segments=N)
    loop_attr = attr_sum / jnp.clip(deg, 1.0, None)[:, None]

    h = jax.nn.elu(_gat_layer(x, src, dst, edge_attr, loop_attr, W1, att_src1,
                              att_dst1, We1, att_edge1, b1, True))
    h = jax.nn.elu(_gat_layer(h, src, dst, edge_attr, loop_attr, W2, att_src2,
                              att_dst2, We2, att_edge2, b2, False))

    onehot = (batch[None, :] == jnp.arange(G, dtype=batch.dtype)[:, None])
    onehot = onehot.astype(jnp.float32)
    cnt = onehot.sum(axis=1)
    bsel = onehot / jnp.clip(cnt, 1.0, None)[:, None]
    stride = N // G
    usel = u[batch[::stride]]

    out = pl.pallas_call(
        _mlp_pool_kernel,
        out_shape=jax.ShapeDtypeStruct((G, 1), jnp.float32),
    )(h, bsel, usel, m1w, m1b.reshape(1, -1), m2w, m2b.reshape(1, -1), m3w,
      m3b.reshape(1, -1))
    return out


# speedup vs baseline: 19.9181x; 17.7177x over previous
import functools

import jax
import jax.numpy as jnp
from jax import lax
from jax.experimental import pallas as pl
from jax.experimental.pallas import tpu as pltpu
from jax.experimental.pallas import tpu_sc as plsc

N = 10000
E = 320000
DF = 128
DE = 16
DG = 16
G = 100
H = 8
C = 64

NSC = 2
NTILE = 16
NW = NSC * NTILE
KF = 128
EPAD = 323584
EXTRA = 256
EPT_A = EPAD // NW
NCH_A = EPT_A // KF
EPT_B = EPAD // NTILE
NCH_B = EPT_B // KF
NP = 10240
RPT = NP // NTILE

_SC_PARAMS = pltpu.CompilerParams(use_tc_tiling_on_sc=False,
                                  needs_layout_passes=False)


def _leaky(x):
    return jnp.maximum(x, 0.2 * x)


def _make_pass_a(with_attr):
    mesh = plsc.VectorSubcoreMesh(core_axis_name="c", subcore_axis_name="s")
    out_type = [
        jax.ShapeDtypeStruct((EPAD + EXTRA, 16), jnp.float32),
        jax.ShapeDtypeStruct((NSC * NP, 16), jnp.float32),
    ]
    scratch = [
        pltpu.VMEM_SHARED((NP, 16), jnp.float32),
        pltpu.VMEM((16,), jnp.float32),
    ]
    for _ in range(2):
        scratch += [
            pltpu.VMEM((KF,), jnp.int32),
            pltpu.VMEM((KF,), jnp.int32),
            pltpu.VMEM((KF, 16), jnp.float32),
            pltpu.VMEM((KF, 16), jnp.float32),
            pltpu.VMEM((KF, 16), jnp.float32),
            pltpu.VMEM((KF, 16), jnp.float32),
            pltpu.SemaphoreType.DMA,
            pltpu.SemaphoreType.DMA,
        ]
    if with_attr:
        out_type.append(jax.ShapeDtypeStruct((NSC * NP, 16), jnp.float32))
        scratch.append(pltpu.VMEM_SHARED((NP, 16), jnp.float32))
        scratch.append(pltpu.VMEM((KF, 16), jnp.float32))
        scratch.append(pltpu.VMEM((KF, 16), jnp.float32))

    def body(s16, d16, ae16, src, dst, m16, z16, ea, n_out, den_out, *rest):
        if with_attr:
            attr_out = rest[0]
            rest = rest[1:]
        den_sh, m_v = rest[0], rest[1]
        slots = [rest[2:10], rest[10:18]]
        if with_attr:
            attr_sh = rest[18]
            eav = [rest[19], rest[20]]

        cid = lax.axis_index("c")
        sid = lax.axis_index("s")
        wid = cid * NTILE + sid
        r0 = sid * RPT

        pltpu.sync_copy(z16.at[pl.ds(r0, RPT)], den_sh.at[pl.ds(r0, RPT)])
        if with_attr:
            pltpu.sync_copy(z16.at[pl.ds(r0, RPT)], attr_sh.at[pl.ds(r0, RPT)])
        pltpu.sync_copy(m16, m_v)
        plsc.subcore_barrier()

        lane = lax.broadcasted_iota(jnp.int32, (16,), 0)
        degrow = jnp.where(lane == 8, 1.0, 0.0)
        mv = m_v[...]
        e0 = wid * EPT_A

        def prefetch(b, ci):
            idxs, idxd, sv, dv, aev, nv, semS, semD = slots[b]
            base = e0 + ci * KF
            pltpu.sync_copy(src.at[pl.ds(base, KF)], idxs)
            pltpu.sync_copy(dst.at[pl.ds(base, KF)], idxd)
            pltpu.sync_copy(ae16.at[pl.ds(base, KF)], aev)
            if with_attr:
                pltpu.sync_copy(ea.at[pl.ds(base, KF)], eav[b])
            pltpu.async_copy(s16.at[idxs], sv, semS)
            pltpu.async_copy(d16.at[idxd], dv, semD)

        def process(b, ci):
            idxs, idxd, sv, dv, aev, nv, semS, semD = slots[b]
            base = e0 + ci * KF
            pltpu.make_async_copy(s16.at[idxs], sv, semS).wait()
            pltpu.make_async_copy(d16.at[idxd], dv, semD).wait()

            def row(i, carry):
                t = sv[i, :] + dv[i, :] + aev[i, :]
                g = dv[i, :] + mv
                val = jnp.exp(_leaky(t) - _leaky(g))
                nv[i, :] = jnp.where(lane < 8, val, degrow)
                return carry

            lax.fori_loop(0, KF, row, 0, unroll=4)
            pltpu.sync_copy(nv, n_out.at[pl.ds(base, KF)])
            pltpu.sync_copy(nv, den_sh.at[idxd], add=True)
            if with_attr:
                pltpu.sync_copy(eav[b], attr_sh.at[idxd], add=True)

        prefetch(0, 0)
        prefetch(1, 1)

        def pair(ci, carry):
            for b in range(2):
                process(b, 2 * ci + b)
                prefetch(b, 2 * ci + b + 2)
            return carry

        lax.fori_loop(0, (NCH_A - 1) // 2, pair, 0)
        process(0, NCH_A - 1)
        _, _, sv1, dv1, _, _, semS1, semD1 = slots[1]
        idxs1, idxd1 = slots[1][0], slots[1][1]
        pltpu.make_async_copy(s16.at[idxs1], sv1, semS1).wait()
        pltpu.make_async_copy(d16.at[idxd1], dv1, semD1).wait()

        plsc.subcore_barrier()
        dump0 = cid * NP + sid * RPT
        pltpu.sync_copy(den_sh.at[pl.ds(sid * RPT, RPT)],
                        den_out.at[pl.ds(dump0, RPT)])
        if with_attr:
            pltpu.sync_copy(attr_sh.at[pl.ds(sid * RPT, RPT)],
                            attr_out.at[pl.ds(dump0, RPT)])

    return pl.kernel(body, out_type=out_type, mesh=mesh,
                     scratch_types=scratch, compiler_params=_SC_PARAMS)


_pass_a_attr = _make_pass_a(True)
_pass_a_plain = _make_pass_a(False)


def _pass_b_body(hp, n16, src, dst, z128, out_hbm, out_sh, *slots_flat):
    slots = [slots_flat[0:6], slots_flat[6:12]]
    cid = lax.axis_index("c")
    sid = lax.axis_index("s")
    r0 = sid * RPT
    e0 = sid * EPT_B

    for p in range(2):
        pp = cid * 2 + p
        h0 = pp * 2
        goff = pp * N
        off = pp * NP
        pltpu.sync_copy(z128.at[pl.ds(r0, RPT)], out_sh.at[pl.ds(r0, RPT)])
        plsc.subcore_barrier()

        def prefetch(b, ci):
            idxs, idxd, nvb, rows, semG, _ = slots[b]
            base = e0 + ci * KF
            pltpu.sync_copy(src.at[pl.ds(base, KF)], idxs)
            pltpu.sync_copy(dst.at[pl.ds(base, KF)], idxd)
            pltpu.sync_copy(n16.at[pl.ds(base, KF)], nvb)
            gv = jnp.broadcast_to(goff, (16,))
            for j in range(KF // 16):
                sl = pl.ds(j * 16, 16)
                idxs[sl] = idxs[sl] + gv
            pltpu.async_copy(hp.at[idxs], rows, semG)

        hv0 = jnp.broadcast_to(h0, (16,))
        hv1 = jnp.broadcast_to(h0 + 1, (16,))

        def process(b, ci):
            idxs, idxd, nvb, rows, semG, _ = slots[b]
            pltpu.make_async_copy(hp.at[idxs], rows, semG).wait()

            def scale(i, carry):
                iv = jnp.broadcast_to(i, (16,))
                s0 = plsc.load_gather(nvb, [iv, hv0])
                s1 = plsc.load_gather(nvb, [iv, hv1])
                for q in range(8):
                    sl = pl.ds(q * 16, 16)
                    rows[i, sl] = rows[i, sl] * (s0 if q < 4 else s1)
                return carry

            lax.fori_loop(0, KF, scale, 0, unroll=4)
            pltpu.sync_copy(rows, out_sh.at[idxd], add=True)

        prefetch(0, 0)
        prefetch(1, 1)

        def pair_iter(ci, carry):
            for b in range(2):
                process(b, 2 * ci + b)
                prefetch(b, 2 * ci + b + 2)
            return carry

        lax.fori_loop(0, NCH_B // 2, pair_iter, 0)
        for b in range(2):
            idxs, idxd, nvb, rows, semG, _ = slots[b]
            pltpu.make_async_copy(hp.at[idxs], rows, semG).wait()

        plsc.subcore_barrier()
        pltpu.sync_copy(out_sh.at[pl.ds(r0, RPT)],
                        out_hbm.at[pl.ds(off + r0, RPT)])
        plsc.subcore_barrier()


def _pass_b_slot_scratch():
    return [
        pltpu.VMEM((KF,), jnp.int32),
        pltpu.VMEM((KF,), jnp.int32),
        pltpu.VMEM((KF, 16), jnp.float32),
        pltpu.VMEM((KF, 2 * C), jnp.float32),
        pltpu.SemaphoreType.DMA,
        pltpu.SemaphoreType.DMA,
    ]


_pass_b = pl.kernel(
    _pass_b_body,
    out_type=[jax.ShapeDtypeStruct((4 * NP, 2 * C), jnp.float32)],
    mesh=plsc.VectorSubcoreMesh(core_axis_name="c", subcore_axis_name="s"),
    scratch_types=[pltpu.VMEM_SHARED((NP, 2 * C), jnp.float32)]
    + _pass_b_slot_scratch() + _pass_b_slot_scratch(),
    compiler_params=_SC_PARAMS,
)


def _mlp_pool_kernel(h_ref, bsel_ref, usel_ref, w1_ref, b1_ref, w2_ref, b2_ref,
                     w3_ref, b3_ref, out_ref):
    ge = bsel_ref[...] @ h_ref[...]
    comb = jnp.concatenate([ge, usel_ref[...]], axis=1)
    z = jnp.maximum(comb @ w1_ref[...] + b1_ref[...], 0.0)
    z = jnp.maximum(z @ w2_ref[...] + b2_ref[...], 0.0)
    out_ref[...] = z @ w3_ref[...] + b3_ref[...]


def _gat_layer(x_in, srcp, dstp, ae_pad, loop_attr, W, att_src, att_dst, We,
               att_edge, bias, concat, z16, z128, with_attr, edge_attr_p):
    Vs = jnp.einsum("fhc,hc->fh", W.reshape(-1, H, C), att_src[0])
    Vd = jnp.einsum("fhc,hc->fh", W.reshape(-1, H, C), att_dst[0])
    Ve = jnp.einsum("dhc,hc->dh", We.reshape(DE, H, C), att_edge[0])

    h = x_in @ W
    al_s = x_in @ Vs
    al_d = x_in @ Vd
    al_e = ae_pad @ Ve

    M = al_s.max(0) + jnp.maximum(al_e.max(0), 0.0)
    s16 = jnp.concatenate([al_s, al_s], axis=1)
    d16 = jnp.pad(jnp.concatenate([al_d, al_d], axis=1),
                  ((0, NP - N), (0, 0)))
    ae16 = jnp.concatenate([al_e, al_e], axis=1)
    m16 = jnp.concatenate([M, M])

    if with_attr:
        n16, den2, attr2 = _pass_a_attr(s16, d16, ae16, srcp, dstp, m16, z16,
                                        edge_attr_p)
        attr_sum = attr2.reshape(NSC, NP, 16)[:, :N].sum(0)
    else:
        n16, den2 = _pass_a_plain(s16, d16, ae16, srcp, dstp, m16, z16,
                                  edge_attr_p)
        attr_sum = None
    den = den2.reshape(NSC, NP, 16)[:, :N].sum(0)
    denom, deg = den[:, :H], den[:, H]

    hp = h.reshape(N, H // 2, 2 * C).transpose(1, 0, 2).reshape(
        (H // 2) * N, 2 * C)
    (outp,) = _pass_b(hp, n16, srcp, dstp, z128)
    outscat = outp.reshape(H // 2, NP, 2, C)[:, :N].transpose(
        1, 0, 2, 3).reshape(N, H, C)

    if loop_attr is None:
        loop_attr = attr_sum / jnp.clip(deg, 1.0, None)[:, None]
    al_e_loop = loop_attr @ Ve
    stab = _leaky(al_d + M[None, :])
    n_self = jnp.exp(_leaky(al_s + al_d + al_e_loop) - stab)

    h3 = h.reshape(N, H, C)
    out = ((outscat + h3 * n_self[:, :, None])
           / (denom + n_self)[:, :, None])
    if concat:
        out = out.reshape(N, H * C) + bias
    else:
        out = out.mean(axis=1) + bias
    return jax.nn.elu(out), loop_attr


def kernel(x, edge_index, edge_attr, u, batch, W1, att_src1, att_dst1, We1,
           att_edge1, b1, W2, att_src2, att_dst2, We2, att_edge2, b2, m1w,
           m1b, m2w, m2b, m3w, m3b):
    src, dst = edge_index[0], edge_index[1]
    npad = EPAD + EXTRA - E
    srcp = jnp.concatenate([src, jnp.zeros((npad,), src.dtype)])
    dstp = jnp.concatenate([dst, jnp.full((npad,), NP - 1, dst.dtype)])
    ae_pad = jnp.pad(edge_attr, ((0, npad), (0, 0)))
    z16 = jnp.zeros((NP, 16), jnp.float32)
    z128 = jnp.zeros((NP, 2 * C), jnp.float32)

    h, loop_attr = _gat_layer(x, srcp, dstp, ae_pad, None, W1, att_src1,
                              att_dst1, We1, att_edge1, b1, True, z16, z128,
                              True, ae_pad)
    h, _ = _gat_layer(h, srcp, dstp, ae_pad, loop_attr, W2, att_src2,
                      att_dst2, We2, att_edge2, b2, False, z16, z128,
                      False, ae_pad)

    onehot = (batch[None, :] == jnp.arange(G, dtype=batch.dtype)[:, None])
    onehot = onehot.astype(jnp.float32)
    cnt = onehot.sum(axis=1)
    bsel = onehot / jnp.clip(cnt, 1.0, None)[:, None]
    stride = N // G
    usel = u[batch[::stride]]

    out = pl.pallas_call(
        _mlp_pool_kernel,
        out_shape=jax.ShapeDtypeStruct((G, 1), jnp.float32),
    )(h, bsel, usel, m1w, m1b.reshape(1, -1), m2w, m2b.reshape(1, -1), m3w,
      m3b.reshape(1, -1))
    return out
